# baseline (speedup 1.0000x reference)
import functools

import jax
import jax.numpy as jnp
from jax import lax
from jax.experimental import pallas as pl
from jax.experimental.pallas import tpu as pltpu
from jax.experimental.pallas import tpu_sc as plsc

N = 8192
K = 192
CUTOFF2 = 0.12 * 0.12
NSUB = 32
ROWS = N // NSUB
LANES = 16
CHUNKS = N // LANES
UNROLL = 1
NROWS = 16
GROUPS = ROWS // NROWS
KR = K + LANES
OUTW = ROWS * KR
BUF = OUTW + LANES


def _body(pos_ref, out_ref, pmax_ref, xs, ys, zs, outbuf, tmpv):
    wid = lax.axis_index("c") * 16 + lax.axis_index("s")
    base = wid * ROWS

    pltpu.sync_copy(pos_ref.at[pl.ds(0, N)], xs)
    pltpu.sync_copy(pos_ref.at[pl.ds(N, N)], ys)
    pltpu.sync_copy(pos_ref.at[pl.ds(2 * N, N)], zs)

    iota = lax.iota(jnp.int32, 16)
    neg1 = jnp.full((LANES,), -1, jnp.int32)
    c2v = jnp.full((LANES,), CUTOFF2, jnp.float32)
    zerov = jnp.zeros((LANES,), jnp.int32)

    def row_body(r, maxcnt):
        ivs, cxs, cys, czs, offs, lims = [], [], [], [], [], []
        for k in range(NROWS):
            rk = k * GROUPS + r
            iv = jnp.full((LANES,), base + rk, jnp.int32)
            ivs.append(iv)
            cxs.append(plsc.load_gather(xs, [iv]))
            cys.append(plsc.load_gather(ys, [iv]))
            czs.append(plsc.load_gather(zs, [iv]))
            off = rk * KR
            offs.append(off)
            lims.append(off + K)
            for kk in range(K // LANES):
                outbuf[pl.ds(off + kk * LANES, LANES)] = neg1

        def block(q, carry):
            jv = carry[-1]
            cnts = list(carry[:-1])
            off0 = q * (UNROLL * LANES)
            for u in range(UNROLL):
                sl = pl.ds(off0 + u * LANES, LANES)
                xv = xs[sl]
                yv = ys[sl]
                zv = zs[sl]
                ms = []
                for k in range(NROWS):
                    dx = xv - cxs[k]
                    dy = yv - cys[k]
                    dz = zv - czs[k]
                    d2 = dx * dx + dy * dy + dz * dz
                    ms.append((d2 <= c2v) & (jv != ivs[k]))
                for k in range(NROWS):
                    dst = jnp.minimum(cnts[k][0], lims[k])
                    plsc.store_compressed(
                        outbuf.at[pl.ds(dst, LANES)], jv, mask=ms[k])
                for k in range(NROWS):
                    cnts[k] = cnts[k] + plsc.all_reduce_population_count(ms[k])
                jv = jv + 16
            return (*cnts, jv)

        init = tuple(jnp.full((LANES,), offs[k], jnp.int32)
                     for k in range(NROWS)) + (iota,)
        res = lax.fori_loop(0, CHUNKS // UNROLL, block, init)
        for k in range(NROWS):
            maxcnt = jnp.maximum(maxcnt, res[k] - offs[k])
        return maxcnt

    maxv = lax.fori_loop(0, GROUPS, row_body, zerov)
    pltpu.sync_copy(outbuf.at[pl.ds(0, OUTW)], out_ref.at[pl.ds(wid * OUTW, OUTW)])
    tmpv[...] = maxv
    pltpu.sync_copy(tmpv, pmax_ref.at[pl.ds(wid * LANES, LANES)])


@jax.jit
def _neigh(pos_t):
    mesh = plsc.VectorSubcoreMesh(core_axis_name="c", subcore_axis_name="s")
    return pl.kernel(
        _body,
        out_type=[
            jax.ShapeDtypeStruct((N * KR,), jnp.int32),
            jax.ShapeDtypeStruct((NSUB * LANES,), jnp.int32),
        ],
        mesh=mesh,
        compiler_params=pltpu.CompilerParams(needs_layout_passes=False),
        scratch_types=[
            pltpu.VMEM((N,), jnp.float32),
            pltpu.VMEM((N,), jnp.float32),
            pltpu.VMEM((N,), jnp.float32),
            pltpu.VMEM((BUF,), jnp.int32),
            pltpu.VMEM((LANES,), jnp.int32),
        ],
    )(pos_t)


def kernel(positions, max_neighbours):
    positions = jnp.asarray(positions)
    pos_t = positions.T.reshape(-1)
    raw, pmax = _neigh(pos_t)
    mn = jnp.asarray(max_neighbours, jnp.int32)
    to_idx = raw.reshape(N, KR)[:, :K] + (mn - K)
    cell_indices = jnp.zeros((N, K, 3), jnp.int32)
    actual_max_neighbours = jnp.max(pmax)
    return to_idx, cell_indices, actual_max_neighbours


# speedup vs baseline: 1.3836x; 1.3836x over previous
import functools

import jax
import jax.numpy as jnp
from jax import lax
from jax.experimental import pallas as pl
from jax.experimental.pallas import tpu as pltpu
from jax.experimental.pallas import tpu_sc as plsc

N = 8192
K = 192
CUTOFF2 = 0.12 * 0.12
NSUB = 32
ROWS = N // NSUB
LANES = 16
CHUNKS = N // LANES
UNROLL = 4
NROWS = 8
GROUPS = ROWS // NROWS
KR = K + LANES
OUTW = ROWS * KR
BUF = OUTW + LANES


def _body(pos_ref, out_ref, pmax_ref, xs, ys, zs, outbuf, tmpv):
    wid = lax.axis_index("c") * 16 + lax.axis_index("s")
    base = wid * ROWS

    pltpu.sync_copy(pos_ref.at[pl.ds(0, N)], xs)
    pltpu.sync_copy(pos_ref.at[pl.ds(N, N)], ys)
    pltpu.sync_copy(pos_ref.at[pl.ds(2 * N, N)], zs)

    iota = lax.iota(jnp.int32, 16)
    neg1 = jnp.full((LANES,), -1, jnp.int32)
    c2v = jnp.full((LANES,), CUTOFF2, jnp.float32)
    zerov = jnp.zeros((LANES,), jnp.int32)

    def row_body(r, maxcnt):
        ivs, cxs, cys, czs, offs, lims = [], [], [], [], [], []
        for k in range(NROWS):
            rk = k * GROUPS + r
            iv = jnp.full((LANES,), base + rk, jnp.int32)
            ivs.append(iv)
            cxs.append(plsc.load_gather(xs, [iv]))
            cys.append(plsc.load_gather(ys, [iv]))
            czs.append(plsc.load_gather(zs, [iv]))
            off = rk * KR
            offs.append(off)
            lims.append(off + K)
            for kk in range(K // LANES):
                outbuf[pl.ds(off + kk * LANES, LANES)] = neg1

        def block(q, carry):
            jv = carry[-1]
            cnts = list(carry[:-1])
            off0 = q * (UNROLL * LANES)
            for u in range(UNROLL):
                sl = pl.ds(off0 + u * LANES, LANES)
                xv = xs[sl]
                yv = ys[sl]
                zv = zs[sl]
                ms = []
                for k in range(NROWS):
                    dx = xv - cxs[k]
                    dy = yv - cys[k]
                    dz = zv - czs[k]
                    d2 = dx * dx + dy * dy + dz * dz
                    ms.append((d2 <= c2v) & (jv != ivs[k]))
                for k in range(NROWS):
                    dst = jnp.minimum(cnts[k][0], lims[k])
                    plsc.store_compressed(
                        outbuf.at[pl.ds(dst, LANES)], jv, mask=ms[k])
                for k in range(NROWS):
                    cnts[k] = cnts[k] + plsc.all_reduce_population_count(ms[k])
                jv = jv + 16
            return (*cnts, jv)

        init = tuple(jnp.full((LANES,), offs[k], jnp.int32)
                     for k in range(NROWS)) + (iota,)
        res = lax.fori_loop(0, CHUNKS // UNROLL, block, init)
        for k in range(NROWS):
            maxcnt = jnp.maximum(maxcnt, res[k] - offs[k])
        return maxcnt

    maxv = lax.fori_loop(0, GROUPS, row_body, zerov)
    pltpu.sync_copy(outbuf.at[pl.ds(0, OUTW)], out_ref.at[pl.ds(wid * OUTW, OUTW)])
    tmpv[...] = maxv
    pltpu.sync_copy(tmpv, pmax_ref.at[pl.ds(wid * LANES, LANES)])


@jax.jit
def _neigh(pos_t):
    mesh = plsc.VectorSubcoreMesh(core_axis_name="c", subcore_axis_name="s")
    return pl.kernel(
        _body,
        out_type=[
            jax.ShapeDtypeStruct((N * KR,), jnp.int32),
            jax.ShapeDtypeStruct((NSUB * LANES,), jnp.int32),
        ],
        mesh=mesh,
        compiler_params=pltpu.CompilerParams(needs_layout_passes=False),
        scratch_types=[
            pltpu.VMEM((N,), jnp.float32),
            pltpu.VMEM((N,), jnp.float32),
            pltpu.VMEM((N,), jnp.float32),
            pltpu.VMEM((BUF,), jnp.int32),
            pltpu.VMEM((LANES,), jnp.int32),
        ],
    )(pos_t)


def kernel(positions, max_neighbours):
    positions = jnp.asarray(positions)
    pos_t = positions.T.reshape(-1)
    raw, pmax = _neigh(pos_t)
    mn = jnp.asarray(max_neighbours, jnp.int32)
    to_idx = raw.reshape(N, KR)[:, :K] + (mn - K)
    cell_indices = jnp.zeros((N, K, 3), jnp.int32)
    actual_max_neighbours = jnp.max(pmax)
    return to_idx, cell_indices, actual_max_neighbours


# speedup vs baseline: 1.4243x; 1.0294x over previous
import jax
import jax.numpy as jnp
from jax import lax
from jax.experimental import pallas as pl
from jax.experimental.pallas import tpu as pltpu
from jax.experimental.pallas import tpu_sc as plsc

N = 8192
K = 192
CUTOFF2 = 0.12 * 0.12
NSUB = 32
ROWS = N // NSUB
LANES = 16
CHUNKS = N // LANES
UNROLL = 2
NROWS = 8
GROUPS = ROWS // NROWS
KR = K + LANES
OUTW = ROWS * KR
BUF = OUTW + LANES


def _body(pos_ref, out_ref, pmax_ref, xs, ys, zs, outbuf, tmpv):
    wid = lax.axis_index("c") * 16 + lax.axis_index("s")
    base = wid * ROWS

    pltpu.sync_copy(pos_ref.at[pl.ds(0, N)], xs)
    pltpu.sync_copy(pos_ref.at[pl.ds(N, N)], ys)
    pltpu.sync_copy(pos_ref.at[pl.ds(2 * N, N)], zs)

    iota = lax.iota(jnp.int32, 16)
    neg1 = jnp.full((LANES,), -1, jnp.int32)
    c2v = jnp.full((LANES,), CUTOFF2, jnp.float32)
    zerov = jnp.zeros((LANES,), jnp.int32)

    def row_body(r, maxcnt):
        ivs, cxs, cys, czs, offs, lims = [], [], [], [], [], []
        for k in range(NROWS):
            rk = k * GROUPS + r
            iv = jnp.full((LANES,), base + rk, jnp.int32)
            ivs.append(iv)
            cxs.append(plsc.load_gather(xs, [iv]))
            cys.append(plsc.load_gather(ys, [iv]))
            czs.append(plsc.load_gather(zs, [iv]))
            off = rk * KR
            offs.append(off)
            lims.append(off + K)
            for kk in range(K // LANES):
                outbuf[pl.ds(off + kk * LANES, LANES)] = neg1

        def block(q, carry):
            jv = carry[-1]
            cnts = list(carry[:-1])
            off0 = q * (UNROLL * LANES)
            for u in range(UNROLL):
                sl = pl.ds(off0 + u * LANES, LANES)
                xv = xs[sl]
                yv = ys[sl]
                zv = zs[sl]
                ms = []
                for k in range(NROWS):
                    dx = xv - cxs[k]
                    dy = yv - cys[k]
                    dz = zv - czs[k]
                    d2 = dx * dx + dy * dy + dz * dz
                    ms.append((d2 <= c2v) & (jv != ivs[k]))
                for k in range(NROWS):
                    dst = jnp.minimum(cnts[k][0], lims[k])
                    plsc.store_compressed(
                        outbuf.at[pl.ds(dst, LANES)], jv, mask=ms[k])
                for k in range(NROWS):
                    cnts[k] = cnts[k] + plsc.all_reduce_population_count(ms[k])
                jv = jv + 16
            return (*cnts, jv)

        init = tuple(jnp.full((LANES,), offs[k], jnp.int32)
                     for k in range(NROWS)) + (iota,)
        res = lax.fori_loop(0, CHUNKS // UNROLL, block, init)
        for k in range(NROWS):
            maxcnt = jnp.maximum(maxcnt, res[k] - offs[k])
        return maxcnt

    maxv = lax.fori_loop(0, GROUPS, row_body, zerov)
    pltpu.sync_copy(outbuf.at[pl.ds(0, OUTW)], out_ref.at[pl.ds(wid * OUTW, OUTW)])
    tmpv[...] = maxv
    pltpu.sync_copy(tmpv, pmax_ref.at[pl.ds(wid * LANES, LANES)])


@jax.jit
def _neigh(pos_t):
    mesh = plsc.VectorSubcoreMesh(core_axis_name="c", subcore_axis_name="s")
    return pl.kernel(
        _body,
        out_type=[
            jax.ShapeDtypeStruct((N * KR,), jnp.int32),
            jax.ShapeDtypeStruct((NSUB * LANES,), jnp.int32),
        ],
        mesh=mesh,
        compiler_params=pltpu.CompilerParams(needs_layout_passes=False),
        scratch_types=[
            pltpu.VMEM((N,), jnp.float32),
            pltpu.VMEM((N,), jnp.float32),
            pltpu.VMEM((N,), jnp.float32),
            pltpu.VMEM((BUF,), jnp.int32),
            pltpu.VMEM((LANES,), jnp.int32),
        ],
    )(pos_t)


def kernel(positions, max_neighbours):
    positions = jnp.asarray(positions)
    pos_t = positions.T.reshape(-1)
    raw, pmax = _neigh(pos_t)
    mn = jnp.asarray(max_neighbours, jnp.int32)
    to_idx = raw.reshape(N, KR)[:, :K] + (mn - K)
    cell_indices = jnp.zeros((N, K, 3), jnp.int32)
    actual_max_neighbours = jnp.max(pmax)
    return to_idx, cell_indices, actual_max_neighbours
